# baseline (speedup 1.0000x reference)
import jax
import jax.numpy as jnp
import numpy as np
from jax import lax
from jax.experimental import pallas as pl
from jax.experimental.pallas import tpu as pltpu
from jax.experimental.pallas import tpu_sc as plsc

VOCAB_SIZE = 100000
D_MODEL = 768
MAX_LEN = 4096
BATCH = 4
SEQ_LEN = 4096

NC = 2
NS = 16
NW = NC * NS
POS_PER_W = SEQ_LEN // NW
P = 32
N_CHUNK = POS_PER_W // P
N_TASK = N_CHUNK * BATCH
NQ = MAX_LEN // P
LANES = 16


def _angle_tables():
    k = np.arange(D_MODEL // 2, dtype=np.float64)
    w = np.power(10000.0, -2.0 * k / D_MODEL)
    q = np.arange(NQ, dtype=np.float64)[:, None]
    r = np.arange(P, dtype=np.float64)[:, None]
    coarse = (P * q) * w[None, :]
    fine = r * w[None, :]

    def interleave(ev, od):
        out = np.empty((ev.shape[0], D_MODEL), dtype=np.float64)
        out[:, 0::2] = ev
        out[:, 1::2] = od
        return out.astype(np.float32)

    a = interleave(np.sin(coarse), np.cos(coarse))
    a2 = interleave(np.cos(coarse), -np.sin(coarse))
    bc = interleave(np.cos(fine), np.cos(fine))
    bs = interleave(np.sin(fine), np.sin(fine))
    return a, a2, bc, bs


_A_NP, _A2_NP, _BC_NP, _BS_NP = _angle_tables()


def _embed_kernel(tab_hbm, idx_hbm, a_hbm, a2_hbm, bc_hbm, bs_hbm, out_hbm,
                  idx_v, a_v, a2_v, bc_v, bs_v, pos_v, tok0, tok1,
                  gsem0, gsem1, ssem0, ssem1, psem):
    wid = lax.axis_index("s") * NC + lax.axis_index("c")
    pos_base = wid * POS_PER_W

    toks = [tok0, tok1]
    gsems = [gsem0, gsem1]
    ssems = [ssem0, ssem1]

    icp = []
    for b in range(BATCH):
        icp.append(pltpu.async_copy(
            idx_hbm.at[b, pl.ds(pos_base, POS_PER_W)],
            idx_v.at[pl.ds(b * POS_PER_W, POS_PER_W)], psem))
    for cp in icp:
        cp.wait()

    def start_gather(t):
        c, b = divmod(t, BATCH)
        isl = idx_v.at[pl.ds(b * POS_PER_W + c * P, P)]
        return pltpu.async_copy(tab_hbm.at[isl], toks[t % 2], gsems[t % 2])

    g_cp = [None] * N_TASK
    s_cp = [None] * N_TASK
    g_cp[0] = start_gather(0)

    tcp = [
        pltpu.async_copy(a_hbm.at[pl.ds(N_CHUNK * wid, N_CHUNK)], a_v, psem),
        pltpu.async_copy(a2_hbm.at[pl.ds(N_CHUNK * wid, N_CHUNK)], a2_v, psem),
        pltpu.async_copy(bc_hbm, bc_v, psem),
        pltpu.async_copy(bs_hbm, bs_v, psem),
    ]
    for cp in tcp:
        cp.wait()

    def compute_pos(c):
        def row(r, carry):
            for j in range(D_MODEL // LANES):
                sl = pl.ds(j * LANES, LANES)
                pos_v[r, sl] = (a_v[c, sl] * bc_v[r, sl]
                                + a2_v[c, sl] * bs_v[r, sl])
            return carry
        lax.fori_loop(0, P, row, 0)

    def add_pos(tok):
        def add_row(r, carry):
            for j in range(D_MODEL // LANES):
                sl = pl.ds(j * LANES, LANES)
                tok[r, sl] = tok[r, sl] + pos_v[r, sl]
            return carry
        lax.fori_loop(0, P, add_row, 0)

    compute_pos(0)
    for t in range(N_TASK):
        c, b = divmod(t, BATCH)
        if t + 1 < N_TASK:
            if t >= 1:
                s_cp[t - 1].wait()
            g_cp[t + 1] = start_gather(t + 1)
        if b == 0 and c > 0:
            compute_pos(c)
        g_cp[t].wait()
        add_pos(toks[t % 2])
        s_cp[t] = pltpu.async_copy(
            toks[t % 2],
            out_hbm.at[pl.ds(b * SEQ_LEN + pos_base + c * P, P)],
            ssems[t % 2])
    s_cp[N_TASK - 2].wait()
    s_cp[N_TASK - 1].wait()


@jax.jit
def _embed(x, tok_table, a, a2, bc, bs):
    x_i32 = x.astype(jnp.int32)
    mesh = plsc.VectorSubcoreMesh(core_axis_name="c", subcore_axis_name="s")
    run = pl.kernel(
        _embed_kernel,
        out_type=jax.ShapeDtypeStruct((BATCH * SEQ_LEN, D_MODEL), jnp.float32),
        mesh=mesh,
        scratch_types=[
            pltpu.VMEM((BATCH * POS_PER_W,), jnp.int32),
            pltpu.VMEM((N_CHUNK, D_MODEL), jnp.float32),
            pltpu.VMEM((N_CHUNK, D_MODEL), jnp.float32),
            pltpu.VMEM((P, D_MODEL), jnp.float32),
            pltpu.VMEM((P, D_MODEL), jnp.float32),
            pltpu.VMEM((P, D_MODEL), jnp.float32),
            pltpu.VMEM((P, D_MODEL), jnp.float32),
            pltpu.VMEM((P, D_MODEL), jnp.float32),
            pltpu.SemaphoreType.DMA,
            pltpu.SemaphoreType.DMA,
            pltpu.SemaphoreType.DMA,
            pltpu.SemaphoreType.DMA,
            pltpu.SemaphoreType.DMA,
        ],
    )
    out = run(tok_table, x_i32, a, a2, bc, bs)
    return out.reshape(BATCH, SEQ_LEN, D_MODEL)


def kernel(x, tok_table):
    return _embed(x, tok_table, _A_NP, _A2_NP, _BC_NP, _BS_NP)


# speedup vs baseline: 1.4625x; 1.4625x over previous
import jax
import jax.numpy as jnp
import numpy as np
from jax import lax
from jax.experimental import pallas as pl
from jax.experimental.pallas import tpu as pltpu
from jax.experimental.pallas import tpu_sc as plsc

VOCAB_SIZE = 100000
D_MODEL = 768
MAX_LEN = 4096
BATCH = 4
SEQ_LEN = 4096

NC = 2
NS = 16
NW = NC * NS
POS_PER_W = SEQ_LEN // NW
P = 32
N_CHUNK = POS_PER_W // P
N_TASK = N_CHUNK * BATCH
LANES = 16


def _sinusoidal_pos_encoding(max_len, d_model):
    pos = np.arange(max_len, dtype=np.float32)[:, None]
    i = np.arange(0, d_model, 2, dtype=np.float32)[None, :]
    angle = pos / np.power(10000.0, i / d_model)
    enc = np.zeros((max_len, d_model), dtype=np.float32)
    enc[:, 0::2] = np.sin(angle)
    enc[:, 1::2] = np.cos(angle)
    return enc


_POS_ENC_NP = _sinusoidal_pos_encoding(MAX_LEN, D_MODEL)


def _embed_kernel(tab_hbm, idx_hbm, pos_hbm, out_hbm,
                  idx_v, pos0, pos1, tok0, tok1,
                  gsem0, gsem1, ssem0, ssem1, psem):
    wid = lax.axis_index("s") * NC + lax.axis_index("c")
    pos_base = wid * POS_PER_W

    toks = [tok0, tok1]
    gsems = [gsem0, gsem1]
    ssems = [ssem0, ssem1]
    poss = [pos0, pos1]

    icp = []
    for b in range(BATCH):
        icp.append(pltpu.async_copy(
            idx_hbm.at[b, pl.ds(pos_base, POS_PER_W)],
            idx_v.at[pl.ds(b * POS_PER_W, POS_PER_W)], psem))
    for cp in icp:
        cp.wait()

    pltpu.sync_copy(pos_hbm.at[pl.ds(pos_base, P)], pos0)

    def start_gather(t):
        c, b = divmod(t, BATCH)
        isl = idx_v.at[pl.ds(b * POS_PER_W + c * P, P)]
        return pltpu.async_copy(tab_hbm.at[isl], toks[t % 2], gsems[t % 2])

    def add_pos(tok, posb):
        def add_row(r, carry):
            for j in range(D_MODEL // LANES):
                sl = pl.ds(j * LANES, LANES)
                tok[r, sl] = tok[r, sl] + posb[r, sl]
            return carry
        lax.fori_loop(0, P, add_row, 0)

    g_cp = [None] * N_TASK
    s_cp = [None] * N_TASK
    p_cp = [None] * N_CHUNK

    g_cp[0] = start_gather(0)
    for t in range(N_TASK):
        c, b = divmod(t, BATCH)
        if b == 0 and c + 1 < N_CHUNK:
            p_cp[c + 1] = pltpu.async_copy(
                pos_hbm.at[pl.ds(pos_base + (c + 1) * P, P)],
                poss[(c + 1) % 2], psem)
        if t + 1 < N_TASK:
            if t >= 1:
                s_cp[t - 1].wait()
            g_cp[t + 1] = start_gather(t + 1)
        g_cp[t].wait()
        if b == 0 and c > 0:
            p_cp[c].wait()
        add_pos(toks[t % 2], poss[c % 2])
        s_cp[t] = pltpu.async_copy(
            toks[t % 2],
            out_hbm.at[pl.ds(b * SEQ_LEN + pos_base + c * P, P)],
            ssems[t % 2])
    s_cp[N_TASK - 2].wait()
    s_cp[N_TASK - 1].wait()


@jax.jit
def _embed(x, tok_table, pos_enc):
    x_i32 = x.astype(jnp.int32)
    mesh = plsc.VectorSubcoreMesh(core_axis_name="c", subcore_axis_name="s")
    run = pl.kernel(
        _embed_kernel,
        out_type=jax.ShapeDtypeStruct((BATCH * SEQ_LEN, D_MODEL), jnp.float32),
        mesh=mesh,
        scratch_types=[
            pltpu.VMEM((BATCH * POS_PER_W,), jnp.int32),
            pltpu.VMEM((P, D_MODEL), jnp.float32),
            pltpu.VMEM((P, D_MODEL), jnp.float32),
            pltpu.VMEM((P, D_MODEL), jnp.float32),
            pltpu.VMEM((P, D_MODEL), jnp.float32),
            pltpu.SemaphoreType.DMA,
            pltpu.SemaphoreType.DMA,
            pltpu.SemaphoreType.DMA,
            pltpu.SemaphoreType.DMA,
            pltpu.SemaphoreType.DMA,
        ],
    )
    out = run(tok_table, x_i32, pos_enc)
    return out.reshape(BATCH, SEQ_LEN, D_MODEL)


def kernel(x, tok_table):
    return _embed(x, tok_table, _POS_ENC_NP)


# speedup vs baseline: 1.4631x; 1.0003x over previous
import jax
import jax.numpy as jnp
import numpy as np
from jax import lax
from jax.experimental import pallas as pl
from jax.experimental.pallas import tpu as pltpu
from jax.experimental.pallas import tpu_sc as plsc

VOCAB_SIZE = 100000
D_MODEL = 768
MAX_LEN = 4096
BATCH = 4
SEQ_LEN = 4096

NC = 2
NS = 16
NW = NC * NS
POS_PER_W = SEQ_LEN // NW
P = 32
N_CHUNK = POS_PER_W // P
N_TASK = N_CHUNK * BATCH
LANES = 16


def _sinusoidal_pos_encoding(max_len, d_model):
    pos = np.arange(max_len, dtype=np.float32)[:, None]
    i = np.arange(0, d_model, 2, dtype=np.float32)[None, :]
    angle = pos / np.power(10000.0, i / d_model)
    enc = np.zeros((max_len, d_model), dtype=np.float32)
    enc[:, 0::2] = np.sin(angle)
    enc[:, 1::2] = np.cos(angle)
    return enc


_POS_ENC_NP = _sinusoidal_pos_encoding(MAX_LEN, D_MODEL)


def _embed_kernel(tab_hbm, idx_hbm, pos_hbm, out_hbm,
                  idx_v, pos0, pos1, tok0, tok1,
                  gsem0, gsem1, ssem0, ssem1, psem):
    wid = lax.axis_index("s") * NC + lax.axis_index("c")
    pos_base = wid * POS_PER_W

    toks = [tok0, tok1]
    gsems = [gsem0, gsem1]
    ssems = [ssem0, ssem1]
    poss = [pos0, pos1]

    icp = []
    for b in range(BATCH):
        icp.append(pltpu.async_copy(
            idx_hbm.at[b, pl.ds(pos_base, POS_PER_W)],
            idx_v.at[pl.ds(b * POS_PER_W, POS_PER_W)], psem))
    for cp in icp:
        cp.wait()

    pltpu.sync_copy(pos_hbm.at[pl.ds(pos_base, P)], pos0)

    def start_gather(t):
        c, b = divmod(t, BATCH)
        isl = idx_v.at[pl.ds(b * POS_PER_W + c * P, P)]
        return pltpu.async_copy(tab_hbm.at[isl], toks[t % 2], gsems[t % 2])

    def add_pos_half(tok, posb, h):
        def add_row(r, carry):
            for j in range(D_MODEL // LANES):
                sl = pl.ds(j * LANES, LANES)
                tok[r, sl] = tok[r, sl] + posb[r, sl]
            return carry
        lax.fori_loop(h * (P // 2), (h + 1) * (P // 2), add_row, 0)

    g_cp = [None] * N_TASK
    s_cp = [None] * N_TASK
    p_cp = [None] * N_CHUNK

    g_cp[0] = start_gather(0)
    for t in range(N_TASK):
        c, b = divmod(t, BATCH)
        if b == 0 and c + 1 < N_CHUNK:
            p_cp[c + 1] = pltpu.async_copy(
                pos_hbm.at[pl.ds(pos_base + (c + 1) * P, P)],
                poss[(c + 1) % 2], psem)
        if t + 1 < N_TASK:
            if t >= 1:
                for cp in s_cp[t - 1]:
                    cp.wait()
            g_cp[t + 1] = start_gather(t + 1)
        g_cp[t].wait()
        if b == 0 and c > 0:
            p_cp[c].wait()
        row0 = b * SEQ_LEN + pos_base + c * P
        add_pos_half(toks[t % 2], poss[c % 2], 0)
        s_half = pltpu.async_copy(
            toks[t % 2].at[pl.ds(0, P // 2)],
            out_hbm.at[pl.ds(row0, P // 2)],
            ssems[t % 2])
        add_pos_half(toks[t % 2], poss[c % 2], 1)
        s_cp[t] = (s_half, pltpu.async_copy(
            toks[t % 2].at[pl.ds(P // 2, P // 2)],
            out_hbm.at[pl.ds(row0 + P // 2, P // 2)],
            ssems[t % 2]))
    for cp in s_cp[N_TASK - 2]:
        cp.wait()
    for cp in s_cp[N_TASK - 1]:
        cp.wait()


@jax.jit
def _embed(x, tok_table, pos_enc):
    x_i32 = x.astype(jnp.int32)
    mesh = plsc.VectorSubcoreMesh(core_axis_name="c", subcore_axis_name="s")
    run = pl.kernel(
        _embed_kernel,
        out_type=jax.ShapeDtypeStruct((BATCH * SEQ_LEN, D_MODEL), jnp.float32),
        mesh=mesh,
        scratch_types=[
            pltpu.VMEM((BATCH * POS_PER_W,), jnp.int32),
            pltpu.VMEM((P, D_MODEL), jnp.float32),
            pltpu.VMEM((P, D_MODEL), jnp.float32),
            pltpu.VMEM((P, D_MODEL), jnp.float32),
            pltpu.VMEM((P, D_MODEL), jnp.float32),
            pltpu.SemaphoreType.DMA,
            pltpu.SemaphoreType.DMA,
            pltpu.SemaphoreType.DMA,
            pltpu.SemaphoreType.DMA,
            pltpu.SemaphoreType.DMA,
        ],
    )
    out = run(tok_table, x_i32, pos_enc)
    return out.reshape(BATCH, SEQ_LEN, D_MODEL)


def kernel(x, tok_table):
    return _embed(x, tok_table, _POS_ENC_NP)
